# Initial kernel scaffold; baseline (speedup 1.0000x reference)
import jax
import jax.numpy as jnp
from jax.experimental import pallas as pl


def kernel(input):
    raise NotImplementedError("write your pallas kernel here")


# speedup vs baseline: 2.2700x; 2.2700x over previous
import jax
import jax.numpy as jnp
from jax.experimental import pallas as pl

_N_ROWS = 4096
_N_FIELDS = 26
_FIELD_SIZE = 1000
_ROW_BLOCK = 128


def _onehot_block(in_ref, out_ref):
    codes = in_ref[...]
    offs = jax.lax.broadcasted_iota(
        jnp.int32, (_ROW_BLOCK, _N_FIELDS, _FIELD_SIZE), 2
    ).astype(jnp.float32)
    out_ref[...] = (offs == codes[:, :, None]).astype(jnp.float32)


def kernel(input):
    n = input.shape[0]
    grid = (n // _ROW_BLOCK,)
    out3 = pl.pallas_call(
        _onehot_block,
        grid=grid,
        in_specs=[pl.BlockSpec((_ROW_BLOCK, _N_FIELDS), lambda r: (r, 0))],
        out_specs=pl.BlockSpec(
            (_ROW_BLOCK, _N_FIELDS, _FIELD_SIZE), lambda r: (r, 0, 0)
        ),
        out_shape=jax.ShapeDtypeStruct(
            (n, _N_FIELDS, _FIELD_SIZE), jnp.float32
        ),
    )(input)
    return out3.reshape(n, _N_FIELDS * _FIELD_SIZE)
